# initial kernel scaffold (unmeasured)
import jax
import jax.numpy as jnp
from jax import lax
from jax.experimental import pallas as pl
from jax.experimental.pallas import tpu as pltpu

N_DEV = 4


def kernel(x, w_mat, scale_x, scale_w):
    m_per, k = x.shape
    n = w_mat.shape[1]
    n_per = n // N_DEV
    my = lax.axis_index("i")

    x8 = x.astype(jnp.float8_e5m2)
    w_blk = lax.dynamic_slice(w_mat, (0, my * n_per), (k, n_per)).astype(
        jnp.float8_e5m2
    )

    def body(x_ref, w_ref, sx_ref, sw_ref, out_ref, comm_ref, send_sems, recv_sems):
        my_pos = lax.axis_index("i")
        left = lax.rem(my_pos + (N_DEV - 1), N_DEV)
        right = lax.rem(my_pos + 1, N_DEV)

        barrier_sem = pltpu.get_barrier_semaphore()
        for nbr in (left, right):
            pl.semaphore_signal(
                barrier_sem, inc=1,
                device_id=(nbr,), device_id_type=pl.DeviceIdType.MESH,
            )
        pl.semaphore_wait(barrier_sem, 2)

        scale = sx_ref[0] * sw_ref[0]

        def gemm_store(chunk, origin):
            acc = jnp.dot(chunk, w_ref[...], preferred_element_type=jnp.float32)
            out_ref[pl.ds(origin * m_per, m_per), :] = jnp.maximum(acc * scale, 0.0)

        comm_ref[0] = x_ref[...]

        for h in range(N_DEV - 1):
            send_slot = h % 2
            recv_slot = (h + 1) % 2
            rdma = pltpu.make_async_remote_copy(
                src_ref=comm_ref.at[send_slot],
                dst_ref=comm_ref.at[recv_slot],
                send_sem=send_sems.at[send_slot],
                recv_sem=recv_sems.at[recv_slot],
                device_id=(right,),
                device_id_type=pl.DeviceIdType.MESH,
            )
            rdma.start()
            if h == 0:
                gemm_store(x_ref[...], my_pos)
            else:
                gemm_store(comm_ref[send_slot], lax.rem(my_pos + (N_DEV - h), N_DEV))
            rdma.wait()
        gemm_store(comm_ref[(N_DEV - 1) % 2], lax.rem(my_pos + 1, N_DEV))

    out_shape = jax.ShapeDtypeStruct((N_DEV * m_per, n_per), jnp.float32)
    return pl.pallas_call(
        body,
        out_shape=out_shape,
        in_specs=[
            pl.BlockSpec(memory_space=pltpu.VMEM),
            pl.BlockSpec(memory_space=pltpu.VMEM),
            pl.BlockSpec(memory_space=pltpu.SMEM),
            pl.BlockSpec(memory_space=pltpu.SMEM),
        ],
        out_specs=pl.BlockSpec(memory_space=pltpu.VMEM),
        scratch_shapes=[
            pltpu.VMEM((2, m_per, k), jnp.float8_e5m2),
            pltpu.SemaphoreType.DMA((2,)),
            pltpu.SemaphoreType.DMA((2,)),
        ],
        compiler_params=pltpu.CompilerParams(collective_id=0),
    )(x8, w_blk, scale_x, scale_w)


# baseline (device time: 217800 ns/iter reference)
import jax
import jax.numpy as jnp
from jax import lax
from jax.experimental import pallas as pl
from jax.experimental.pallas import tpu as pltpu

N_DEV = 4


def kernel(x, w_mat, scale_x, scale_w):
    m_per, k = x.shape
    n = w_mat.shape[1]
    n_per = n // N_DEV
    my = lax.axis_index("i")

    x8 = x.astype(jnp.float8_e5m2)
    w_blk = lax.dynamic_slice(w_mat, (0, my * n_per), (k, n_per)).astype(
        jnp.float8_e5m2
    )

    def body(x_ref, w_ref, sx_ref, sw_ref, out_ref, comm_ref, send_sems, recv_sems):
        my_pos = lax.axis_index("i")
        left = lax.rem(my_pos + (N_DEV - 1), N_DEV)
        right = lax.rem(my_pos + 1, N_DEV)

        barrier_sem = pltpu.get_barrier_semaphore()
        for nbr in (left, right):
            pl.semaphore_signal(
                barrier_sem, inc=1,
                device_id=(nbr,), device_id_type=pl.DeviceIdType.MESH,
            )
        pl.semaphore_wait(barrier_sem, 2)

        scale = sx_ref[0] * sw_ref[0]

        def gemm_store(chunk, origin):
            acc = jnp.dot(chunk, w_ref[...], preferred_element_type=jnp.float32)
            out_ref[pl.ds(origin * m_per, m_per), :] = jnp.maximum(acc * scale, 0.0)

        comm_ref[0] = x_ref[...]

        for h in range(N_DEV - 1):
            send_slot = h % 2
            recv_slot = (h + 1) % 2
            rdma = pltpu.make_async_remote_copy(
                src_ref=comm_ref.at[send_slot],
                dst_ref=comm_ref.at[recv_slot],
                send_sem=send_sems.at[send_slot],
                recv_sem=recv_sems.at[recv_slot],
                device_id=(right,),
                device_id_type=pl.DeviceIdType.MESH,
            )
            rdma.start()
            if h == 0:
                gemm_store(x_ref[...], my_pos)
            else:
                gemm_store(comm_ref[send_slot], lax.rem(my_pos + (N_DEV - h), N_DEV))
            rdma.wait()
        gemm_store(comm_ref[(N_DEV - 1) % 2], lax.rem(my_pos + 1, N_DEV))

    out_shape = jax.ShapeDtypeStruct((N_DEV * m_per, n_per), jnp.float32)
    return pl.pallas_call(
        body,
        out_shape=out_shape,
        in_specs=[
            pl.BlockSpec(memory_space=pltpu.VMEM),
            pl.BlockSpec(memory_space=pltpu.VMEM),
            pl.BlockSpec(memory_space=pltpu.SMEM),
            pl.BlockSpec(memory_space=pltpu.SMEM),
        ],
        out_specs=pl.BlockSpec(memory_space=pltpu.VMEM),
        scratch_shapes=[
            pltpu.VMEM((2, m_per, k), jnp.float8_e5m2),
            pltpu.SemaphoreType.DMA((2,)),
            pltpu.SemaphoreType.DMA((2,)),
        ],
        compiler_params=pltpu.CompilerParams(
            collective_id=0, vmem_limit_bytes=100 * 1024 * 1024
        ),
    )(x8, w_blk, scale_x, scale_w)


# device time: 150393 ns/iter; 1.4482x vs baseline; 1.4482x over previous
import jax
import jax.numpy as jnp
from jax import lax
from jax.experimental import pallas as pl
from jax.experimental.pallas import tpu as pltpu

N_DEV = 4


def kernel(x, w_mat, scale_x, scale_w):
    m_per, k = x.shape
    n = w_mat.shape[1]
    n_per = n // N_DEV
    m_half = m_per // 2
    my = lax.axis_index("i")

    w_blk = lax.dynamic_slice(w_mat, (0, my * n_per), (k, n_per)).astype(
        jnp.float8_e5m2
    )
    x8 = x.astype(jnp.float8_e5m2)

    def body(x_ref, w_ref, sx_ref, sw_ref, out_ref,
             comm_r, comm_l, send_r, recv_r, send_l, recv_l):
        my_pos = lax.axis_index("i")
        left = lax.rem(my_pos + (N_DEV - 1), N_DEV)
        right = lax.rem(my_pos + 1, N_DEV)

        barrier_sem = pltpu.get_barrier_semaphore()
        for nbr in (left, right):
            pl.semaphore_signal(
                barrier_sem, inc=1,
                device_id=(nbr,), device_id_type=pl.DeviceIdType.MESH,
            )
        pl.semaphore_wait(barrier_sem, 2)

        scale = sx_ref[0] * sw_ref[0]

        comm_r[0] = x_ref[: m_half, :]
        comm_l[0] = x_ref[m_half:, :]

        def gemm_store(chunk, origin, row_off):
            acc = jnp.dot(chunk, w_ref[...], preferred_element_type=jnp.float32)
            out_ref[pl.ds(origin * m_per + row_off, m_half), :] = jnp.maximum(
                acc * scale, 0.0
            )

        for h in range(N_DEV - 1):
            rdma_r = pltpu.make_async_remote_copy(
                src_ref=comm_r.at[h],
                dst_ref=comm_r.at[h + 1],
                send_sem=send_r.at[h],
                recv_sem=recv_r.at[h],
                device_id=(right,),
                device_id_type=pl.DeviceIdType.MESH,
            )
            rdma_l = pltpu.make_async_remote_copy(
                src_ref=comm_l.at[h],
                dst_ref=comm_l.at[h + 1],
                send_sem=send_l.at[h],
                recv_sem=recv_l.at[h],
                device_id=(left,),
                device_id_type=pl.DeviceIdType.MESH,
            )
            rdma_r.start()
            rdma_l.start()
            gemm_store(comm_r[h], lax.rem(my_pos + (N_DEV - h), N_DEV), 0)
            gemm_store(comm_l[h], lax.rem(my_pos + h, N_DEV), m_half)
            rdma_r.wait()
            rdma_l.wait()

        gemm_store(comm_r[N_DEV - 1], lax.rem(my_pos + 1, N_DEV), 0)
        gemm_store(comm_l[N_DEV - 1], lax.rem(my_pos + (N_DEV - 1), N_DEV), m_half)

    out_shape = jax.ShapeDtypeStruct((N_DEV * m_per, n_per), jnp.float32)
    return pl.pallas_call(
        body,
        out_shape=out_shape,
        in_specs=[
            pl.BlockSpec(memory_space=pltpu.VMEM),
            pl.BlockSpec(memory_space=pltpu.VMEM),
            pl.BlockSpec(memory_space=pltpu.SMEM),
            pl.BlockSpec(memory_space=pltpu.SMEM),
        ],
        out_specs=pl.BlockSpec(memory_space=pltpu.VMEM),
        scratch_shapes=[
            pltpu.VMEM((N_DEV, m_half, k), jnp.float8_e5m2),
            pltpu.VMEM((N_DEV, m_half, k), jnp.float8_e5m2),
            pltpu.SemaphoreType.DMA((N_DEV - 1,)),
            pltpu.SemaphoreType.DMA((N_DEV - 1,)),
            pltpu.SemaphoreType.DMA((N_DEV - 1,)),
            pltpu.SemaphoreType.DMA((N_DEV - 1,)),
        ],
        compiler_params=pltpu.CompilerParams(
            collective_id=0, vmem_limit_bytes=100 * 1024 * 1024
        ),
    )(x8, w_blk, scale_x, scale_w)


# device time: 116645 ns/iter; 1.8672x vs baseline; 1.2893x over previous
import jax
import jax.numpy as jnp
from jax import lax
from jax.experimental import pallas as pl
from jax.experimental.pallas import tpu as pltpu

N_DEV = 4
KC = 512


def kernel(x, w_mat, scale_x, scale_w):
    m_per, k = x.shape
    n = w_mat.shape[1]
    n_per = n // N_DEV
    m_half = m_per // 2
    n_kc = k // KC

    def body(x_hbm, w_hbm, sx_ref, sw_ref, out_hbm,
             comm_r, comm_l, x_stage, w_stage, w8, acc,
             send_r, recv_r, send_l, recv_l, load_sems, out_sems):
        my_pos = lax.axis_index("i")
        left = lax.rem(my_pos + (N_DEV - 1), N_DEV)
        right = lax.rem(my_pos + 1, N_DEV)
        col0 = my_pos * n_per

        x_cp = pltpu.make_async_copy(x_hbm, x_stage, load_sems.at[0])
        x_cp.start()

        def w_cp(c, slot):
            return pltpu.make_async_copy(
                w_hbm.at[pl.ds(c * KC, KC), pl.ds(col0, n_per)],
                w_stage.at[slot],
                load_sems.at[1 + slot],
            )

        w_cp(0, 0).start()
        w_cp(1, 1).start()

        x_cp.wait()
        comm_r[0] = x_stage[: m_half, :].astype(jnp.float8_e5m2)
        comm_l[0] = x_stage[m_half:, :].astype(jnp.float8_e5m2)

        barrier_sem = pltpu.get_barrier_semaphore()
        for nbr in (left, right):
            pl.semaphore_signal(
                barrier_sem, inc=1,
                device_id=(nbr,), device_id_type=pl.DeviceIdType.MESH,
            )
        pl.semaphore_wait(barrier_sem, 2)

        scale = sx_ref[0] * sw_ref[0]

        out_copies = []

        def gemm_store(s, top):
            b = len(out_copies)
            slot = b % 2
            if b >= 2:
                out_copies[b - 2].wait()
            origin = lax.rem(my_pos + (N_DEV - s if top else s), N_DEV)
            chunk = comm_r[s] if top else comm_l[s]
            a = jnp.dot(chunk, w8[...], preferred_element_type=jnp.float32)
            acc[slot] = jnp.maximum(a * scale, 0.0)
            row0 = origin * m_per + (0 if top else m_half)
            cp = pltpu.make_async_copy(
                acc.at[slot],
                out_hbm.at[pl.ds(row0, m_half), :],
                out_sems.at[slot],
            )
            cp.start()
            out_copies.append(cp)

        for h in range(N_DEV - 1):
            rdma_r = pltpu.make_async_remote_copy(
                src_ref=comm_r.at[h], dst_ref=comm_r.at[h + 1],
                send_sem=send_r.at[h], recv_sem=recv_r.at[h],
                device_id=(right,), device_id_type=pl.DeviceIdType.MESH,
            )
            rdma_l = pltpu.make_async_remote_copy(
                src_ref=comm_l.at[h], dst_ref=comm_l.at[h + 1],
                send_sem=send_l.at[h], recv_sem=recv_l.at[h],
                device_id=(left,), device_id_type=pl.DeviceIdType.MESH,
            )
            rdma_r.start()
            rdma_l.start()
            if h == 0:
                for c in range(n_kc):
                    slot = c % 2
                    w_cp(c, slot).wait()
                    w8[pl.ds(c * KC, KC), :] = w_stage[slot].astype(
                        jnp.float8_e5m2
                    )
                    if c + 2 < n_kc:
                        w_cp(c + 2, slot).start()
            elif h == 1:
                for s, top in ((0, True), (0, False), (1, True), (1, False)):
                    gemm_store(s, top)
            else:
                for s, top in ((2, True), (2, False)):
                    gemm_store(s, top)
            rdma_r.wait()
            rdma_l.wait()

        gemm_store(3, True)
        gemm_store(3, False)
        out_copies[-2].wait()
        out_copies[-1].wait()

    out_shape = jax.ShapeDtypeStruct((N_DEV * m_per, n_per), jnp.float32)
    return pl.pallas_call(
        body,
        out_shape=out_shape,
        in_specs=[
            pl.BlockSpec(memory_space=pl.ANY),
            pl.BlockSpec(memory_space=pl.ANY),
            pl.BlockSpec(memory_space=pltpu.SMEM),
            pl.BlockSpec(memory_space=pltpu.SMEM),
        ],
        out_specs=pl.BlockSpec(memory_space=pl.ANY),
        scratch_shapes=[
            pltpu.VMEM((N_DEV, m_half, k), jnp.float8_e5m2),
            pltpu.VMEM((N_DEV, m_half, k), jnp.float8_e5m2),
            pltpu.VMEM((m_per, k), jnp.float32),
            pltpu.VMEM((2, KC, n_per), jnp.float32),
            pltpu.VMEM((k, n_per), jnp.float8_e5m2),
            pltpu.VMEM((2, m_half, n_per), jnp.float32),
            pltpu.SemaphoreType.DMA((N_DEV - 1,)),
            pltpu.SemaphoreType.DMA((N_DEV - 1,)),
            pltpu.SemaphoreType.DMA((N_DEV - 1,)),
            pltpu.SemaphoreType.DMA((N_DEV - 1,)),
            pltpu.SemaphoreType.DMA((3,)),
            pltpu.SemaphoreType.DMA((2,)),
        ],
        compiler_params=pltpu.CompilerParams(
            collective_id=0, vmem_limit_bytes=100 * 1024 * 1024
        ),
    )(x, w_mat, scale_x, scale_w)


# device time: 105089 ns/iter; 2.0725x vs baseline; 1.1100x over previous
import os

import jax
import jax.numpy as jnp
from jax import lax
from jax.experimental import pallas as pl
from jax.experimental.pallas import tpu as pltpu

N_DEV = 4
KC = 512
KMODE = os.environ.get("KMODE", "full")


def kernel(x, w_mat, scale_x, scale_w):
    m_per, k = x.shape
    n = w_mat.shape[1]
    n_per = n // N_DEV
    m_half = m_per // 2
    n_kc = k // KC

    def body(x_hbm, w_hbm, sx_ref, sw_ref, out_hbm,
             comm_r, comm_l, x_stage, w_stage, w8, acc,
             send_r, recv_r, send_l, recv_l, load_sems, out_sems):
        my_pos = lax.axis_index("i")
        left = lax.rem(my_pos + (N_DEV - 1), N_DEV)
        right = lax.rem(my_pos + 1, N_DEV)
        col0 = my_pos * n_per

        x_cp = pltpu.make_async_copy(x_hbm, x_stage, load_sems.at[0])
        x_cp.start()

        def w_cp(c, slot):
            return pltpu.make_async_copy(
                w_hbm.at[pl.ds(c * KC, KC), pl.ds(col0, n_per)],
                w_stage.at[slot],
                load_sems.at[1 + slot],
            )

        if KMODE != "comm":
            w_cp(0, 0).start()
            w_cp(1, 1).start()

        x_cp.wait()
        comm_r[0] = x_stage[: m_half, :].astype(jnp.float8_e5m2)
        comm_l[0] = x_stage[m_half:, :].astype(jnp.float8_e5m2)

        barrier_sem = pltpu.get_barrier_semaphore()
        for nbr in (left, right):
            pl.semaphore_signal(
                barrier_sem, inc=1,
                device_id=(nbr,), device_id_type=pl.DeviceIdType.MESH,
            )
        pl.semaphore_wait(barrier_sem, 2)

        scale = sx_ref[0] * sw_ref[0]

        out_copies = []

        def gemm_store(s, top):
            b = len(out_copies)
            slot = b % 2
            if b >= 2:
                out_copies[b - 2].wait()
            origin = lax.rem(my_pos + (N_DEV - s if top else s), N_DEV)
            chunk = comm_r[s] if top else comm_l[s]
            a = jnp.dot(chunk, w8[...], preferred_element_type=jnp.float32)
            acc[slot] = jnp.maximum(a * scale, 0.0)
            row0 = origin * m_per + (0 if top else m_half)
            cp = pltpu.make_async_copy(
                acc.at[slot],
                out_hbm.at[pl.ds(row0, m_half), :],
                out_sems.at[slot],
            )
            cp.start()
            out_copies.append(cp)

        for h in range(N_DEV - 1):
            rdma_r = pltpu.make_async_remote_copy(
                src_ref=comm_r.at[h], dst_ref=comm_r.at[h + 1],
                send_sem=send_r.at[h], recv_sem=recv_r.at[h],
                device_id=(right,), device_id_type=pl.DeviceIdType.MESH,
            )
            rdma_l = pltpu.make_async_remote_copy(
                src_ref=comm_l.at[h], dst_ref=comm_l.at[h + 1],
                send_sem=send_l.at[h], recv_sem=recv_l.at[h],
                device_id=(left,), device_id_type=pl.DeviceIdType.MESH,
            )
            if KMODE != "compute":
                rdma_r.start()
                rdma_l.start()
            if KMODE == "comm":
                pass
            elif h == 0:
                for c in range(n_kc):
                    slot = c % 2
                    w_cp(c, slot).wait()
                    w8[pl.ds(c * KC, KC), :] = w_stage[slot].astype(
                        jnp.float8_e5m2
                    )
                    if c + 2 < n_kc:
                        w_cp(c + 2, slot).start()
            elif h == 1:
                for s, top in ((0, True), (0, False), (1, True), (1, False)):
                    gemm_store(s, top)
            else:
                for s, top in ((2, True), (2, False)):
                    gemm_store(s, top)
            if KMODE != "compute":
                rdma_r.wait()
                rdma_l.wait()

        if KMODE != "comm":
            gemm_store(3, True)
            gemm_store(3, False)
            out_copies[-2].wait()
            out_copies[-1].wait()

    out_shape = jax.ShapeDtypeStruct((N_DEV * m_per, n_per), jnp.float32)
    return pl.pallas_call(
        body,
        out_shape=out_shape,
        in_specs=[
            pl.BlockSpec(memory_space=pl.ANY),
            pl.BlockSpec(memory_space=pl.ANY),
            pl.BlockSpec(memory_space=pltpu.SMEM),
            pl.BlockSpec(memory_space=pltpu.SMEM),
        ],
        out_specs=pl.BlockSpec(memory_space=pl.ANY),
        scratch_shapes=[
            pltpu.VMEM((N_DEV, m_half, k), jnp.float8_e5m2),
            pltpu.VMEM((N_DEV, m_half, k), jnp.float8_e5m2),
            pltpu.VMEM((m_per, k), jnp.float32),
            pltpu.VMEM((2, KC, n_per), jnp.float32),
            pltpu.VMEM((k, n_per), jnp.float8_e5m2),
            pltpu.VMEM((2, m_half, n_per), jnp.float32),
            pltpu.SemaphoreType.DMA((N_DEV - 1,)),
            pltpu.SemaphoreType.DMA((N_DEV - 1,)),
            pltpu.SemaphoreType.DMA((N_DEV - 1,)),
            pltpu.SemaphoreType.DMA((N_DEV - 1,)),
            pltpu.SemaphoreType.DMA((3,)),
            pltpu.SemaphoreType.DMA((2,)),
        ],
        compiler_params=pltpu.CompilerParams(
            collective_id=0, vmem_limit_bytes=100 * 1024 * 1024
        ),
    )(x, w_mat, scale_x, scale_w)


# device time: 80050 ns/iter; 2.7208x vs baseline; 1.3128x over previous
import os

import jax
import jax.numpy as jnp
from jax import lax
from jax.experimental import pallas as pl
from jax.experimental.pallas import tpu as pltpu

N_DEV = 4
KC = 512
KMODE = os.environ.get("KMODE", "full")


def kernel(x, w_mat, scale_x, scale_w):
    m_per, k = x.shape
    n = w_mat.shape[1]
    n_per = n // N_DEV
    m_half = m_per // 2
    n_kc = k // KC

    def body(x_hbm, w_hbm, sx_ref, sw_ref, out_hbm,
             comm_r, comm_l, x_stage, w_stage, w8, acc,
             send_r, recv_r, send_l, recv_l, load_sems, out_sems):
        my_pos = lax.axis_index("i")
        left = lax.rem(my_pos + (N_DEV - 1), N_DEV)
        right = lax.rem(my_pos + 1, N_DEV)
        col0 = my_pos * n_per

        x_cp = pltpu.make_async_copy(x_hbm, x_stage, load_sems.at[0])
        x_cp.start()

        def w_cp(c, slot):
            return pltpu.make_async_copy(
                w_hbm.at[pl.ds(c * KC, KC), pl.ds(col0, n_per)],
                w_stage.at[slot],
                load_sems.at[1 + slot],
            )

        if KMODE not in ("comm", "commr"):
            w_cp(0, 0).start()
            w_cp(1, 1).start()

        x_cp.wait()
        comm_r[0] = x_stage[: m_half, :].astype(jnp.float8_e5m2)
        comm_l[0] = x_stage[m_half:, :].astype(jnp.float8_e5m2)

        barrier_sem = pltpu.get_barrier_semaphore()
        for nbr in (left, right):
            pl.semaphore_signal(
                barrier_sem, inc=1,
                device_id=(nbr,), device_id_type=pl.DeviceIdType.MESH,
            )
        pl.semaphore_wait(barrier_sem, 2)

        scale = sx_ref[0] * sw_ref[0]

        out_copies = []

        def gemm_store(s, top):
            b = len(out_copies)
            slot = b % 2
            if b >= 2:
                out_copies[b - 2].wait()
            origin = lax.rem(my_pos + (N_DEV - s if top else s), N_DEV)
            chunk = comm_r[s] if top else comm_l[s]
            a = jnp.dot(chunk, w8[...], preferred_element_type=jnp.float32)
            acc[slot] = jnp.maximum(a * scale, 0.0)
            row0 = origin * m_per + (0 if top else m_half)
            cp = pltpu.make_async_copy(
                acc.at[slot],
                out_hbm.at[pl.ds(row0, m_half), :],
                out_sems.at[slot],
            )
            cp.start()
            out_copies.append(cp)

        for h in range(N_DEV - 1):
            rdma_r = pltpu.make_async_remote_copy(
                src_ref=comm_r.at[h], dst_ref=comm_r.at[h + 1],
                send_sem=send_r.at[h], recv_sem=recv_r.at[h],
                device_id=(right,), device_id_type=pl.DeviceIdType.MESH,
            )
            rdma_l = pltpu.make_async_remote_copy(
                src_ref=comm_l.at[h], dst_ref=comm_l.at[h + 1],
                send_sem=send_l.at[h], recv_sem=recv_l.at[h],
                device_id=(left,), device_id_type=pl.DeviceIdType.MESH,
            )
            if KMODE != "compute":
                rdma_r.start()
                if KMODE != "commr":
                    rdma_l.start()
            if KMODE in ("comm", "commr"):
                pass
            elif h == 0:
                for c in range(n_kc):
                    slot = c % 2
                    w_cp(c, slot).wait()
                    w8[pl.ds(c * KC, KC), :] = w_stage[slot].astype(
                        jnp.float8_e5m2
                    )
                    if c + 2 < n_kc:
                        w_cp(c + 2, slot).start()
            elif h == 1:
                for s, top in ((0, True), (0, False), (1, True), (1, False)):
                    gemm_store(s, top)
            else:
                for s, top in ((2, True), (2, False)):
                    gemm_store(s, top)
            if KMODE != "compute":
                rdma_r.wait()
                if KMODE != "commr":
                    rdma_l.wait()

        if KMODE not in ("comm", "commr"):
            gemm_store(3, True)
            gemm_store(3, False)
            out_copies[-2].wait()
            out_copies[-1].wait()

    out_shape = jax.ShapeDtypeStruct((N_DEV * m_per, n_per), jnp.float32)
    return pl.pallas_call(
        body,
        out_shape=out_shape,
        in_specs=[
            pl.BlockSpec(memory_space=pl.ANY),
            pl.BlockSpec(memory_space=pl.ANY),
            pl.BlockSpec(memory_space=pltpu.SMEM),
            pl.BlockSpec(memory_space=pltpu.SMEM),
        ],
        out_specs=pl.BlockSpec(memory_space=pl.ANY),
        scratch_shapes=[
            pltpu.VMEM((N_DEV, m_half, k), jnp.float8_e5m2),
            pltpu.VMEM((N_DEV, m_half, k), jnp.float8_e5m2),
            pltpu.VMEM((m_per, k), jnp.float32),
            pltpu.VMEM((2, KC, n_per), jnp.float32),
            pltpu.VMEM((k, n_per), jnp.float8_e5m2),
            pltpu.VMEM((2, m_half, n_per), jnp.float32),
            pltpu.SemaphoreType.DMA((N_DEV - 1,)),
            pltpu.SemaphoreType.DMA((N_DEV - 1,)),
            pltpu.SemaphoreType.DMA((N_DEV - 1,)),
            pltpu.SemaphoreType.DMA((N_DEV - 1,)),
            pltpu.SemaphoreType.DMA((3,)),
            pltpu.SemaphoreType.DMA((2,)),
        ],
        compiler_params=pltpu.CompilerParams(
            collective_id=0, vmem_limit_bytes=100 * 1024 * 1024
        ),
    )(x, w_mat, scale_x, scale_w)
